# Initial kernel scaffold; baseline (speedup 1.0000x reference)
import jax
import jax.numpy as jnp
from jax.experimental import pallas as pl


def kernel(Xs, Xt, betas):
    raise NotImplementedError("write your pallas kernel here")


# speedup vs baseline: 3.9846x; 3.9846x over previous
import jax
import jax.numpy as jnp
import numpy as np
from jax.experimental import pallas as pl
from jax.experimental.pallas import tpu as pltpu

_N_KERNELS = 29
_LANES = 128


def _mmd_body(xs_ref, xt_ref, aux_ref, out_ref):
    j = pl.program_id(1)
    xs = xs_ref[...]
    xt = xt_ref[...]
    f = xs.shape[1] // 2
    a_s, b_s = xs[:, :f], xs[:, f:]
    a_t, b_t = xt[:, :f], xt[:, f:]

    def sqd(u, v):
        d = u - v
        return jnp.sum(d * d, axis=1, keepdims=True)

    dxx = sqd(a_s, b_s)
    dyy = sqd(a_t, b_t)
    dxy = sqd(a_s, b_t)
    dyx = sqd(b_s, a_t)

    c = aux_ref[0:1, :]
    w = aux_ref[1:2, :]
    s = (jnp.exp(dxx * c) + jnp.exp(dyy * c)
         - jnp.exp(dxy * c) - jnp.exp(dyx * c))
    part = jnp.sum(s * w)

    @pl.when(j == 0)
    def _():
        out_ref[...] = jnp.zeros_like(out_ref)

    out_ref[...] += part


def kernel(Xs, Xt, betas):
    n, f = Xs.shape
    m = (n // 2) * 2
    p = m // 2
    Xs2 = Xs[:m].reshape(p, 2 * f)
    Xt2 = Xt[:m].reshape(p, 2 * f)

    gammas = np.power(np.float32(2.0),
                      np.arange(-3.5, 3.75, 0.25, dtype=np.float32))
    neg_inv = (-1.0 / (2.0 * gammas * gammas)).astype(np.float32)
    aux = jnp.zeros((8, _LANES), dtype=jnp.float32)
    aux = aux.at[0, :_N_KERNELS].set(jnp.asarray(neg_inv))
    aux = aux.at[1, :_N_KERNELS].set(betas[:, 0] / np.float32(p))

    ni = 8
    blk = 512
    nj = p // (ni * blk)
    assert ni * nj * blk == p, (p, ni, nj, blk)

    out = pl.pallas_call(
        _mmd_body,
        grid=(ni, nj),
        in_specs=[
            pl.BlockSpec((blk, 2 * f), lambda i, j: (i * nj + j, 0)),
            pl.BlockSpec((blk, 2 * f), lambda i, j: (i * nj + j, 0)),
            pl.BlockSpec((8, _LANES), lambda i, j: (0, 0)),
        ],
        out_specs=pl.BlockSpec((1, 1, _LANES), lambda i, j: (i, 0, 0)),
        out_shape=jax.ShapeDtypeStruct((ni, 1, _LANES), jnp.float32),
        compiler_params=pltpu.CompilerParams(
            dimension_semantics=("parallel", "arbitrary"),
        ),
    )(Xs2, Xt2, aux)

    return jnp.sum(out[:, 0, 0]).reshape(1)


# speedup vs baseline: 4.2061x; 1.0556x over previous
import jax
import jax.numpy as jnp
import numpy as np
from jax.experimental import pallas as pl
from jax.experimental.pallas import tpu as pltpu

_N_KERNELS = 29
_LANES = 128


def _mmd_body(xs_ref, xt_ref, aux_ref, out_ref):
    j = pl.program_id(1)
    xs = xs_ref[...]
    xt = xt_ref[...]
    f = xs.shape[1] // 2
    a_s, b_s = xs[:, :f], xs[:, f:]
    a_t, b_t = xt[:, :f], xt[:, f:]

    def sqd(u, v):
        d = u - v
        return jnp.sum(d * d, axis=1, keepdims=True)

    dxx = sqd(a_s, b_s)
    dyy = sqd(a_t, b_t)
    dxy = sqd(a_s, b_t)
    dyx = sqd(b_s, a_t)

    c = aux_ref[0:1, :]
    w = aux_ref[1:2, :]
    s = (jnp.exp(dxx * c) + jnp.exp(dyy * c)
         - jnp.exp(dxy * c) - jnp.exp(dyx * c))
    part = jnp.sum(s * w)

    @pl.when(j == 0)
    def _():
        out_ref[...] = jnp.zeros_like(out_ref)

    out_ref[...] += part


def kernel(Xs, Xt, betas):
    n, f = Xs.shape
    m = (n // 2) * 2
    p = m // 2
    Xs2 = Xs[:m].reshape(p, 2 * f)
    Xt2 = Xt[:m].reshape(p, 2 * f)

    gammas = np.power(np.float32(2.0),
                      np.arange(-3.5, 3.75, 0.25, dtype=np.float32))
    neg_inv = (-1.0 / (2.0 * gammas * gammas)).astype(np.float32)
    aux = jnp.zeros((8, _LANES), dtype=jnp.float32)
    aux = aux.at[0, :_N_KERNELS].set(jnp.asarray(neg_inv))
    aux = aux.at[1, :_N_KERNELS].set(betas[:, 0] / np.float32(p))

    ni = 8
    blk = 2048
    nj = p // (ni * blk)
    assert ni * nj * blk == p, (p, ni, nj, blk)

    out = pl.pallas_call(
        _mmd_body,
        grid=(ni, nj),
        in_specs=[
            pl.BlockSpec((blk, 2 * f), lambda i, j: (i * nj + j, 0)),
            pl.BlockSpec((blk, 2 * f), lambda i, j: (i * nj + j, 0)),
            pl.BlockSpec((8, _LANES), lambda i, j: (0, 0)),
        ],
        out_specs=pl.BlockSpec((1, 1, _LANES), lambda i, j: (i, 0, 0)),
        out_shape=jax.ShapeDtypeStruct((ni, 1, _LANES), jnp.float32),
        compiler_params=pltpu.CompilerParams(
            dimension_semantics=("parallel", "arbitrary"),
        ),
    )(Xs2, Xt2, aux)

    return jnp.sum(out[:, 0, 0]).reshape(1)


# speedup vs baseline: 15.1710x; 3.6069x over previous
import jax
import jax.numpy as jnp
import numpy as np
from jax.experimental import pallas as pl
from jax.experimental.pallas import tpu as pltpu

_N_KERNELS = 29
_LANES = 128


def _mmd_body(xs_ref, xt_ref, aux_ref, out_ref):
    j = pl.program_id(1)
    xs = xs_ref[...]
    xt = xt_ref[...]
    nrows = xs.shape[0]
    xs_n = pltpu.roll(xs, nrows - 1, 0)
    xt_n = pltpu.roll(xt, nrows - 1, 0)

    def sqd(u, v):
        d = u - v
        return jnp.sum(d * d, axis=1, keepdims=True)

    dxx = sqd(xs, xs_n)
    dyy = sqd(xt, xt_n)
    dxy = sqd(xs, xt_n)
    dyx = sqd(xt, xs_n)

    c = aux_ref[0:1, :]
    w = aux_ref[1:2, :]
    s = (jnp.exp(dxx * c) + jnp.exp(dyy * c)
         - jnp.exp(dxy * c) - jnp.exp(dyx * c))

    rows = jax.lax.broadcasted_iota(jnp.int32, s.shape, 0)
    even = (rows % 2) == 0
    part = jnp.sum(jnp.where(even, s * w, 0.0))

    @pl.when(j == 0)
    def _():
        out_ref[...] = jnp.zeros_like(out_ref)

    out_ref[...] += part


def kernel(Xs, Xt, betas):
    n, f = Xs.shape
    m = (n // 2) * 2
    p = m // 2

    gammas = np.power(np.float32(2.0),
                      np.arange(-3.5, 3.75, 0.25, dtype=np.float32))
    neg_inv = (-1.0 / (2.0 * gammas * gammas)).astype(np.float32)
    aux = jnp.zeros((8, _LANES), dtype=jnp.float32)
    aux = aux.at[0, :_N_KERNELS].set(jnp.asarray(neg_inv))
    aux = aux.at[1, :_N_KERNELS].set(betas[:, 0] / np.float32(p))

    ni = 8
    blk = 2048
    nj = p // (ni * blk)
    assert ni * nj * blk == p, (p, ni, nj, blk)

    out = pl.pallas_call(
        _mmd_body,
        grid=(ni, nj),
        in_specs=[
            pl.BlockSpec((2 * blk, f), lambda i, j: (i * nj + j, 0)),
            pl.BlockSpec((2 * blk, f), lambda i, j: (i * nj + j, 0)),
            pl.BlockSpec((8, _LANES), lambda i, j: (0, 0)),
        ],
        out_specs=pl.BlockSpec((1, 1, _LANES), lambda i, j: (i, 0, 0)),
        out_shape=jax.ShapeDtypeStruct((ni, 1, _LANES), jnp.float32),
        compiler_params=pltpu.CompilerParams(
            dimension_semantics=("parallel", "arbitrary"),
        ),
    )(Xs[:m], Xt[:m], aux)

    return jnp.sum(out[:, 0, 0]).reshape(1)


# speedup vs baseline: 15.9515x; 1.0514x over previous
import jax
import jax.numpy as jnp
import numpy as np
from jax.experimental import pallas as pl
from jax.experimental.pallas import tpu as pltpu

_N_KERNELS = 29
_LANES = 128


def _mmd_body(xs_ref, xt_ref, aux_ref, out_ref):
    j = pl.program_id(1)
    xs = xs_ref[...]
    xt = xt_ref[...]
    xs_n = pltpu.roll(xs, 7, 1)
    xt_n = pltpu.roll(xt, 7, 1)

    def sqd(u, v):
        d = u - v
        return jnp.sum(d * d, axis=2, keepdims=True)

    dxx = sqd(xs, xs_n)
    dyy = sqd(xt, xt_n)
    dxy = sqd(xs, xt_n)
    dyx = sqd(xt, xs_n)

    c = aux_ref[0:1, :].reshape(1, 1, _LANES)
    w = aux_ref[1:2, :].reshape(1, 1, _LANES)
    s = (jnp.exp(dxx * c) + jnp.exp(dyy * c)
         - jnp.exp(dxy * c) - jnp.exp(dyx * c))

    rows = jax.lax.broadcasted_iota(jnp.int32, s.shape, 1)
    even = (rows % 2) == 0
    part = jnp.sum(jnp.where(even, s * w, 0.0))

    @pl.when(j == 0)
    def _():
        out_ref[...] = jnp.zeros_like(out_ref)

    out_ref[...] += part


def kernel(Xs, Xt, betas):
    n, f = Xs.shape
    m = (n // 8) * 8
    p = m // 2

    gammas = np.power(np.float32(2.0),
                      np.arange(-3.5, 3.75, 0.25, dtype=np.float32))
    neg_inv = (-1.0 / (2.0 * gammas * gammas)).astype(np.float32)
    aux = jnp.zeros((8, _LANES), dtype=jnp.float32)
    aux = aux.at[0, :_N_KERNELS].set(jnp.asarray(neg_inv))
    aux = aux.at[1, :_N_KERNELS].set(betas[:, 0] / np.float32(p))

    ni = 8
    b8 = 512
    nj = m // (ni * b8 * 8)
    assert ni * nj * b8 * 8 == m, (m, ni, nj, b8)

    out = pl.pallas_call(
        _mmd_body,
        grid=(ni, nj),
        in_specs=[
            pl.BlockSpec((b8, 8, f), lambda i, j: (i * nj + j, 0, 0)),
            pl.BlockSpec((b8, 8, f), lambda i, j: (i * nj + j, 0, 0)),
            pl.BlockSpec((8, _LANES), lambda i, j: (0, 0)),
        ],
        out_specs=pl.BlockSpec((1, 1, _LANES), lambda i, j: (i, 0, 0)),
        out_shape=jax.ShapeDtypeStruct((ni, 1, _LANES), jnp.float32),
        compiler_params=pltpu.CompilerParams(
            dimension_semantics=("parallel", "arbitrary"),
        ),
    )(Xs[:m].reshape(m // 8, 8, f), Xt[:m].reshape(m // 8, 8, f), aux)

    return jnp.sum(out[:, 0, 0]).reshape(1)
